# baseline (speedup 1.0000x reference)
import functools

import jax
import jax.numpy as jnp
from jax import lax
from jax.experimental import pallas as pl
from jax.experimental.pallas import tpu as pltpu
from jax.experimental.pallas import tpu_sc as plsc


def kernel(rel_pos_emb, seq_len):
    del seq_len
    V, D = rel_pos_emb.shape
    N = (V + 1) // 2

    info = plsc.get_sparse_core_info()
    NC, NS = info.num_cores, info.num_subcores
    NW = NC * NS
    rpw = N // NW

    mesh = plsc.VectorSubcoreMesh(core_axis_name="c", subcore_axis_name="s")
    blk = N * D

    @functools.partial(
        pl.kernel,
        mesh=mesh,
        out_type=jax.ShapeDtypeStruct((N * N * D,), jnp.float32),
        scratch_types=[pltpu.SemaphoreType.DMA],
    )
    def sliding_copy(table_hbm, out_hbm, sem):
        c = lax.axis_index("c")
        s = lax.axis_index("s")
        wid = s * NC + c
        copies = []
        for t in range(rpw):
            i = wid * rpw + t
            copies.append(
                pltpu.async_copy(
                    table_hbm.at[pl.ds((N - 1 - i) * D, blk)],
                    out_hbm.at[pl.ds(i * blk, blk)],
                    sem,
                )
            )
        for cp in copies:
            cp.wait()

    return sliding_copy(rel_pos_emb.reshape(-1)).reshape(N, N, D)


# speedup vs baseline: 21.1462x; 21.1462x over previous
import functools

import jax
import jax.numpy as jnp
from jax import lax
from jax.experimental import pallas as pl
from jax.experimental.pallas import tpu as pltpu
from jax.experimental.pallas import tpu_sc as plsc


def kernel(rel_pos_emb, seq_len):
    del seq_len
    V, D = rel_pos_emb.shape
    N = (V + 1) // 2

    info = plsc.get_sparse_core_info()
    NC, NS = info.num_cores, info.num_subcores
    NW = NC * NS
    rpw = N // NW

    mesh = plsc.VectorSubcoreMesh(core_axis_name="c", subcore_axis_name="s")
    JC = N // 2
    win = JC + rpw - 1
    blk = JC * D

    @functools.partial(
        pl.kernel,
        mesh=mesh,
        out_type=jax.ShapeDtypeStruct((N * N * D,), jnp.float32),
        scratch_types=[
            pltpu.VMEM((win * D,), jnp.float32),
            pltpu.SemaphoreType.DMA,
        ],
    )
    def sliding_copy(table_hbm, out_hbm, buf, sem):
        c = lax.axis_index("c")
        s = lax.axis_index("s")
        wid = s * NC + c
        i0 = wid * rpw
        for j0 in (0, JC):
            base = (N - rpw - i0 + j0) * D
            pltpu.sync_copy(table_hbm.at[pl.ds(base, win * D)], buf)
            copies = []
            for t in range(rpw):
                copies.append(
                    pltpu.async_copy(
                        buf.at[pl.ds((rpw - 1 - t) * D, blk)],
                        out_hbm.at[pl.ds(((i0 + t) * N + j0) * D, blk)],
                        sem,
                    )
                )
            for cp in copies:
                cp.wait()

    return sliding_copy(rel_pos_emb.reshape(-1)).reshape(N, N, D)


# speedup vs baseline: 21.1749x; 1.0014x over previous
import functools

import jax
import jax.numpy as jnp
from jax import lax
from jax.experimental import pallas as pl
from jax.experimental.pallas import tpu as pltpu
from jax.experimental.pallas import tpu_sc as plsc


def kernel(rel_pos_emb, seq_len):
    del seq_len
    V, D = rel_pos_emb.shape
    N = (V + 1) // 2

    info = plsc.get_sparse_core_info()
    NC, NS = info.num_cores, info.num_subcores
    NW = NC * NS
    rpw = N // NW

    mesh = plsc.VectorSubcoreMesh(core_axis_name="c", subcore_axis_name="s")
    JC = N // 2
    win = JC + rpw

    @functools.partial(
        pl.kernel,
        mesh=mesh,
        out_type=jax.ShapeDtypeStruct((N, N, D), jnp.float32),
        scratch_types=[
            pltpu.VMEM((win, D), jnp.float32),
            pltpu.SemaphoreType.DMA,
        ],
        compiler_params=pltpu.CompilerParams(use_tc_tiling_on_sc=False),
    )
    def sliding_copy(table_hbm, out_hbm, buf, sem):
        c = lax.axis_index("c")
        s = lax.axis_index("s")
        wid = s * NC + c
        i0 = wid * rpw
        for j0 in (0, JC):
            base = pl.multiple_of(N - rpw - i0 + j0, 8)
            pltpu.sync_copy(table_hbm.at[pl.ds(base, win)], buf)
            copies = []
            for t in range(rpw):
                copies.append(
                    pltpu.async_copy(
                        buf.at[pl.ds(rpw - 1 - t, JC)],
                        out_hbm.at[i0 + t, pl.ds(j0, JC)],
                        sem,
                    )
                )
            for cp in copies:
                cp.wait()

    table = jnp.concatenate([rel_pos_emb, rel_pos_emb[-1:]], axis=0)
    return sliding_copy(table)


# speedup vs baseline: 61.3136x; 2.8956x over previous
import functools

import jax
import jax.numpy as jnp
from jax import lax
from jax.experimental import pallas as pl
from jax.experimental.pallas import tpu as pltpu
from jax.experimental.pallas import tpu_sc as plsc


def kernel(rel_pos_emb, seq_len):
    del seq_len
    V, D = rel_pos_emb.shape
    N = (V + 1) // 2

    info = plsc.get_sparse_core_info()
    NC, NS = info.num_cores, info.num_subcores
    NW = NC * NS
    rpw = N // NW
    NG = NW // 8
    JC = N // 2
    win = JC + 8 * (rpw - 1)

    mesh = plsc.VectorSubcoreMesh(core_axis_name="c", subcore_axis_name="s")

    @functools.partial(
        pl.kernel,
        mesh=mesh,
        out_type=jax.ShapeDtypeStruct((N, N, D), jnp.float32),
        scratch_types=[
            pltpu.VMEM((win, D), jnp.float32),
            pltpu.SemaphoreType.DMA,
        ],
    )
    def sliding_copy(t8_hbm, out_hbm, buf, sem):
        c = lax.axis_index("c")
        s = lax.axis_index("s")
        wid = s * NC + c
        rcls = wid % 8
        g = wid // 8
        i_base = 7 - rcls + 8 * rpw * g
        b_last = N - 8 - 8 * (rpw * g + rpw - 1)
        for j0 in (0, JC):
            base = pl.multiple_of(b_last + j0, 8)
            pltpu.sync_copy(t8_hbm.at[rcls, pl.ds(base, win)], buf)
            copies = []
            for m in range(rpw):
                off = 8 * (rpw - 1 - m)
                copies.append(
                    pltpu.async_copy(
                        buf.at[pl.ds(off, JC)],
                        out_hbm.at[i_base + 8 * m, pl.ds(j0, JC)],
                        sem,
                    )
                )
            for cp in copies:
                cp.wait()

    rows = jnp.clip(jnp.arange(2 * N)[None, :] + jnp.arange(8)[:, None], 0, V - 1)
    t8 = rel_pos_emb[rows]
    return sliding_copy(t8)


# speedup vs baseline: 69.7136x; 1.1370x over previous
import functools

import jax
import jax.numpy as jnp
from jax import lax
from jax.experimental import pallas as pl
from jax.experimental.pallas import tpu as pltpu
from jax.experimental.pallas import tpu_sc as plsc


def kernel(rel_pos_emb, seq_len):
    del seq_len
    V, D = rel_pos_emb.shape
    N = (V + 1) // 2

    info = plsc.get_sparse_core_info()
    NC, NS = info.num_cores, info.num_subcores
    NW = NC * NS
    rpw = N // NW
    NG = NW // 8
    JC = N // 2
    win = JC + 8 * (rpw - 1)

    mesh = plsc.VectorSubcoreMesh(core_axis_name="c", subcore_axis_name="s")

    @functools.partial(
        pl.kernel,
        mesh=mesh,
        out_type=jax.ShapeDtypeStruct((N, N, D), jnp.float32),
        scratch_types=[
            pltpu.VMEM((win, D), jnp.float32),
            pltpu.SemaphoreType.DMA,
        ],
    )
    def sliding_copy(t8_hbm, out_hbm, buf, sem):
        c = lax.axis_index("c")
        s = lax.axis_index("s")
        wid = s * NC + c
        rcls = wid % 8
        g = wid // 8
        i_base = 7 - rcls + 8 * rpw * g
        b_last = N - 8 - 8 * (rpw * g + rpw - 1)
        for j0 in (0, JC):
            base = pl.multiple_of(b_last + j0, 8)
            pltpu.sync_copy(t8_hbm.at[rcls, pl.ds(base, win)], buf)
            copies = []
            for m in range(rpw):
                off = 8 * (rpw - 1 - m)
                copies.append(
                    pltpu.async_copy(
                        buf.at[pl.ds(off, JC)],
                        out_hbm.at[i_base + 8 * m, pl.ds(j0, JC)],
                        sem,
                    )
                )
            for cp in copies:
                cp.wait()

    pad = jnp.concatenate(
        [rel_pos_emb, jnp.broadcast_to(rel_pos_emb[-1:], (2 * N + 7 - V, D))]
    )
    t8 = jnp.stack([lax.slice_in_dim(pad, s, s + 2 * N) for s in range(8)])
    return sliding_copy(t8)
